# baseline (device time: 182994 ns/iter reference)
import jax
import jax.numpy as jnp
from jax import lax
from jax.experimental import pallas as pl
from jax.experimental.pallas import tpu as pltpu

N_DEV = 16
N_HOPS = 8

RING = (1, 5, 9, 13, 14, 10, 6, 2, 3, 7, 11, 15, 12, 8, 4, 0)
INV = tuple(RING.index(i) for i in range(N_DEV))


def _lookup(table, idx):
    v = jnp.int32(table[0])
    for j in range(1, len(table)):
        v = jnp.where(idx == j, jnp.int32(table[j]), v)
    return v


def kernel(x, w_mat):
    m_per, k = x.shape
    n_per = w_mat.shape[1]
    m_half = m_per // 2

    def body(x_ref, w_ref, out_ref, comm_ref, ssr, rsr, ssl, rsl):
        me = lax.axis_index("i")
        r = _lookup(INV, me)
        right = _lookup(RING, (r + 1) % N_DEV)
        left = _lookup(RING, (r - 1) % N_DEV)

        comm_ref[pl.ds(2 * r, 2)] = x_ref[...].astype(jnp.bfloat16).reshape(
            2, m_half, k
        )
        w_bf16 = w_ref[...].astype(jnp.bfloat16)

        barrier_sem = pltpu.get_barrier_semaphore()
        for nbr in (left, right):
            pl.semaphore_signal(
                barrier_sem, inc=1,
                device_id=(nbr,), device_id_type=pl.DeviceIdType.MESH,
            )
        pl.semaphore_wait(barrier_sem, 2)

        def gemm_half(hs):
            off = _lookup(RING, hs // 2) * m_per + (hs % 2) * m_half
            y = jnp.dot(
                comm_ref[hs], w_bf16, preferred_element_type=jnp.float32
            )
            out_ref[pl.ds(off, m_half), :] = jnp.maximum(y, 0.0)

        def make_rdma(hs, sems_s, sems_r, dev):
            return pltpu.make_async_remote_copy(
                src_ref=comm_ref.at[hs],
                dst_ref=comm_ref.at[hs],
                send_sem=sems_s.at[hs],
                recv_sem=sems_r.at[hs],
                device_id=(dev,),
                device_id_type=pl.DeviceIdType.MESH,
            )

        pending_sends = []

        def send(hs, sems_s, sems_r, dev):
            rdma = make_rdma(hs, sems_s, sems_r, dev)
            rdma.start()
            pending_sends.append(rdma)

        send(2 * r, ssr, rsr, right)
        send(2 * r + 1, ssl, rsl, left)
        send(2 * r + 1, ssr, rsr, right)
        send(2 * r, ssl, rsl, left)
        gemm_half(2 * r)
        gemm_half(2 * r + 1)

        for h in range(1, N_HOPS + 1):
            rs = (r - h) % N_DEV
            ls = (r + h) % N_DEV

            make_rdma(2 * rs, ssr, rsr, left).wait_recv()
            if h < N_HOPS:
                send(2 * rs, ssr, rsr, right)
            gemm_half(2 * rs)

            make_rdma(2 * ls + 1, ssl, rsl, right).wait_recv()
            if h < N_HOPS:
                send(2 * ls + 1, ssl, rsl, left)
            gemm_half(2 * ls + 1)

            if h < N_HOPS:
                make_rdma(2 * rs + 1, ssr, rsr, left).wait_recv()
                if h < N_HOPS - 1:
                    send(2 * rs + 1, ssr, rsr, right)
                gemm_half(2 * rs + 1)

                make_rdma(2 * ls, ssl, rsl, right).wait_recv()
                if h < N_HOPS - 1:
                    send(2 * ls, ssl, rsl, left)
                gemm_half(2 * ls)

        for rdma in pending_sends:
            rdma.wait_send()

    return pl.pallas_call(
        body,
        out_shape=jax.ShapeDtypeStruct((N_DEV * m_per, n_per), jnp.float32),
        in_specs=[
            pl.BlockSpec(memory_space=pltpu.VMEM),
            pl.BlockSpec(memory_space=pltpu.VMEM),
        ],
        out_specs=pl.BlockSpec(memory_space=pltpu.VMEM),
        scratch_shapes=[
            pltpu.VMEM((2 * N_DEV, m_half, k), jnp.bfloat16),
            pltpu.SemaphoreType.DMA((2 * N_DEV,)),
            pltpu.SemaphoreType.DMA((2 * N_DEV,)),
            pltpu.SemaphoreType.DMA((2 * N_DEV,)),
            pltpu.SemaphoreType.DMA((2 * N_DEV,)),
        ],
        compiler_params=pltpu.CompilerParams(
            collective_id=0, vmem_limit_bytes=100 * 1024 * 1024
        ),
    )(x, w_mat)
